# baseline (speedup 1.0000x reference)
import functools
import jax
import jax.numpy as jnp
from jax import lax
from jax.experimental import pallas as pl
from jax.experimental.pallas import tpu as pltpu
from jax.experimental.pallas import tpu_sc as plsc

EMB = 64
SCALE = 8.0
NC = 2
NS = 16
NW = NC * NS
G = 128
NBUF = 4
D = 2


@functools.partial(jax.jit, static_argnums=(2,))
def _run(idx3, table, ns):
    mesh = plsc.VectorSubcoreMesh(core_axis_name="c", subcore_axis_name="s")

    @functools.partial(
        pl.kernel,
        mesh=mesh,
        out_type=jax.ShapeDtypeStruct((ns, 8, NW, 8, G), jnp.float32),
        scratch_types=[
            pltpu.VMEM((ns, G), jnp.int32),
            pltpu.VMEM((NBUF, G, EMB), jnp.float32),
            pltpu.VMEM((NBUF, 8, 8, G), jnp.float32),
        ]
        + [pltpu.SemaphoreType.DMA] * (2 * NBUF),
        compiler_params=pltpu.CompilerParams(
            use_tc_tiling_on_sc=False, needs_layout_passes=False
        ),
    )
    def k(idx_hbm, table_hbm, out_hbm, idx_v, rows_v, stg_v, *sems):
        in_sems = sems[:NBUF]
        out_sems = sems[NBUF:]
        wid = lax.axis_index("s") * NC + lax.axis_index("c")
        pltpu.sync_copy(idx_hbm.at[wid], idx_v)

        def gather(s, b):
            return pltpu.async_copy(
                table_hbm.at[idx_v.at[s]], rows_v.at[b], in_sems[b]
            )

        def wait_gather(s, b):
            pltpu.make_async_copy(
                table_hbm.at[idx_v.at[s]], rows_v.at[b], in_sems[b]
            ).wait()

        def put(s, b):
            return pltpu.async_copy(
                stg_v.at[b], out_hbm.at[s, :, wid], out_sems[b]
            )

        def wait_put(b):
            pltpu.make_async_copy(
                stg_v.at[b], out_hbm.at[0, :, wid], out_sems[b]
            ).wait()

        lanes = lax.iota(jnp.int32, 16)

        def xpose_scale(b):
            def body(g, carry):
                for c_lo in range(8):
                    col = jnp.broadcast_to(g * 8 + c_lo, (16,)).astype(jnp.int32)
                    for lc in range(8):
                        row = lc * 16 + lanes
                        v = plsc.load_gather(rows_v.at[b], [row, col])
                        stg_v[b, g, c_lo, pl.ds(lc * 16, 16)] = v * SCALE
                return carry

            lax.fori_loop(0, 8, body, 0)

        gather(0, 0)
        gather(1, 1)
        for s in range(D):
            gather(s + D, s + D)
            wait_gather(s, s)
            xpose_scale(s)
            put(s, s)

        def steady(t, c):
            for b in range(NBUF):
                s = D + t * NBUF + b
                cb = (D + b) % NBUF
                wait_put(b)
                gather(s + D, b)
                wait_gather(s, cb)
                xpose_scale(cb)
                put(s, cb)
            return c

        lax.fori_loop(0, (ns - 2 * D) // NBUF, steady, 0)

        for i in range(D):
            s = ns - D + i
            cb = s % NBUF
            wait_gather(s, cb)
            xpose_scale(cb)
            put(s, cb)
        for b in range(NBUF):
            wait_put(b)

    return k(idx3, table)


def kernel(tokens, table):
    nb, ns = tokens.shape
    idx3 = tokens.astype(jnp.int32).T.reshape(ns, NW, G).transpose(1, 0, 2)
    out5 = _run(idx3, table, ns)
    return jnp.transpose(out5, (2, 4, 0, 1, 3)).reshape(nb, ns, EMB)


# speedup vs baseline: 1.2421x; 1.2421x over previous
import functools
import jax
import jax.numpy as jnp
from jax import lax
from jax.experimental import pallas as pl
from jax.experimental.pallas import tpu as pltpu
from jax.experimental.pallas import tpu_sc as plsc

EMB = 64
SCALE = 8.0
NC = 2
NS = 16
NW = NC * NS
G = 128
NBUF = 4
D = 2


@functools.partial(jax.jit, static_argnums=(2,))
def _run(idx3, table, ns):
    mesh = plsc.VectorSubcoreMesh(core_axis_name="c", subcore_axis_name="s")

    @functools.partial(
        pl.kernel,
        mesh=mesh,
        out_type=jax.ShapeDtypeStruct((ns, 8, NW, 8, G), jnp.float32),
        scratch_types=[
            pltpu.VMEM((ns, G), jnp.int32),
            pltpu.VMEM((NBUF, G, EMB), jnp.float32),
            pltpu.VMEM((NBUF, 8, 8, G), jnp.float32),
        ]
        + [pltpu.SemaphoreType.DMA] * (2 * NBUF),
        compiler_params=pltpu.CompilerParams(
            use_tc_tiling_on_sc=False, needs_layout_passes=False
        ),
    )
    def k(idx_hbm, table_hbm, out_hbm, idx_v, rows_v, stg_v, *sems):
        in_sems = sems[:NBUF]
        out_sems = sems[NBUF:]
        wid = lax.axis_index("s") * NC + lax.axis_index("c")
        pltpu.sync_copy(idx_hbm.at[wid], idx_v)

        def gather(s, b):
            return pltpu.async_copy(
                table_hbm.at[idx_v.at[s]], rows_v.at[b], in_sems[b]
            )

        def wait_gather(s, b):
            pltpu.make_async_copy(
                table_hbm.at[idx_v.at[s]], rows_v.at[b], in_sems[b]
            ).wait()

        def put(s, b):
            return pltpu.async_copy(
                stg_v.at[b], out_hbm.at[s, :, wid], out_sems[b]
            )

        def wait_put(b):
            pltpu.make_async_copy(
                stg_v.at[b], out_hbm.at[0, :, wid], out_sems[b]
            ).wait()

        lanes = lax.iota(jnp.int32, 16)
        rowv = [lc * 16 + lanes for lc in range(8)]

        def xpose_scale(b):
            def body(g, carry):
                for c_lo in range(8):
                    col = jnp.broadcast_to(g * 8 + c_lo, (16,)).astype(jnp.int32)
                    vs = [
                        plsc.load_gather(rows_v.at[b], [rowv[lc], col])
                        for lc in range(8)
                    ]
                    for lc in range(8):
                        stg_v[b, g, c_lo, pl.ds(lc * 16, 16)] = vs[lc] * SCALE
                return carry

            lax.fori_loop(0, 8, body, 0)

        gather(0, 0)
        gather(1, 1)
        for s in range(D):
            gather(s + D, s + D)
            wait_gather(s, s)
            xpose_scale(s)
            put(s, s)

        def steady(t, c):
            for b in range(NBUF):
                s = D + t * NBUF + b
                cb = (D + b) % NBUF
                wait_put(b)
                gather(s + D, b)
                wait_gather(s, cb)
                xpose_scale(cb)
                put(s, cb)
            return c

        lax.fori_loop(0, (ns - 2 * D) // NBUF, steady, 0)

        for i in range(D):
            s = ns - D + i
            cb = s % NBUF
            wait_gather(s, cb)
            xpose_scale(cb)
            put(s, cb)
        for b in range(NBUF):
            wait_put(b)

    return k(idx3, table)


def kernel(tokens, table):
    nb, ns = tokens.shape
    idx3 = tokens.astype(jnp.int32).T.reshape(ns, NW, G).transpose(1, 0, 2)
    out5 = _run(idx3, table, ns)
    return jnp.transpose(out5, (2, 4, 0, 1, 3)).reshape(nb, ns, EMB)


# speedup vs baseline: 2.6595x; 2.1411x over previous
import functools
import jax
import jax.numpy as jnp
from jax import lax
from jax.experimental import pallas as pl
from jax.experimental.pallas import tpu as pltpu
from jax.experimental.pallas import tpu_sc as plsc

EMB = 64
SCALE = 8.0
NC = 2
NS = 16
NW = NC * NS
G = 128
NBUF = 4
D = 2


@functools.partial(jax.jit, static_argnums=(2,))
def _run(idx3, table, ns):
    mesh = plsc.VectorSubcoreMesh(core_axis_name="c", subcore_axis_name="s")

    @functools.partial(
        pl.kernel,
        mesh=mesh,
        out_type=jax.ShapeDtypeStruct((ns, 8, NW, 8, G), jnp.float32),
        scratch_types=[
            pltpu.VMEM((ns, G), jnp.int32),
            pltpu.VMEM((NBUF, G, EMB), jnp.float32),
            pltpu.VMEM((NBUF, 8, 8, G), jnp.float32),
        ]
        + [pltpu.SemaphoreType.DMA] * (2 * NBUF),
        compiler_params=pltpu.CompilerParams(
            use_tc_tiling_on_sc=False, needs_layout_passes=False
        ),
    )
    def k(idx_hbm, table_hbm, out_hbm, idx_v, rows_v, stg_v, *sems):
        in_sems = sems[:NBUF]
        out_sems = sems[NBUF:]
        wid = lax.axis_index("s") * NC + lax.axis_index("c")
        pltpu.sync_copy(idx_hbm.at[wid], idx_v)

        def gather(s, b):
            return pltpu.async_copy(
                table_hbm.at[idx_v.at[s]], rows_v.at[b], in_sems[b]
            )

        def wait_gather(s, b):
            pltpu.make_async_copy(
                table_hbm.at[idx_v.at[s]], rows_v.at[b], in_sems[b]
            ).wait()

        def put(s, b):
            return pltpu.async_copy(
                stg_v.at[b], out_hbm.at[s, :, wid], out_sems[b]
            )

        def wait_put(b):
            pltpu.make_async_copy(
                stg_v.at[b], out_hbm.at[0, :, wid], out_sems[b]
            ).wait()

        lanes = lax.iota(jnp.int32, 16)
        lanes9 = lanes * 9
        rowv = [lc * 16 + lanes for lc in range(8)]

        def xpose_scale(b):
            def body(c0, carry):
                cvec = (c0 + lanes9) & 63
                gvec = cvec >> 3
                clovec = cvec & 7
                vs = [
                    plsc.load_gather(rows_v.at[b], [rowv[lc], cvec])
                    for lc in range(8)
                ]
                for lc in range(8):
                    plsc.store_scatter(
                        stg_v.at[b], [gvec, clovec, rowv[lc]], vs[lc] * SCALE
                    )
                return carry

            lax.fori_loop(0, EMB, body, 0)

        gather(0, 0)
        gather(1, 1)
        for s in range(D):
            gather(s + D, s + D)
            wait_gather(s, s)
            xpose_scale(s)
            put(s, s)

        def steady(t, c):
            for b in range(NBUF):
                s = D + t * NBUF + b
                cb = (D + b) % NBUF
                wait_put(b)
                gather(s + D, b)
                wait_gather(s, cb)
                xpose_scale(cb)
                put(s, cb)
            return c

        lax.fori_loop(0, (ns - 2 * D) // NBUF, steady, 0)

        for i in range(D):
            s = ns - D + i
            cb = s % NBUF
            wait_gather(s, cb)
            xpose_scale(cb)
            put(s, cb)
        for b in range(NBUF):
            wait_put(b)

    return k(idx3, table)


def kernel(tokens, table):
    nb, ns = tokens.shape
    idx3 = tokens.astype(jnp.int32).T.reshape(ns, NW, G).transpose(1, 0, 2)
    out5 = _run(idx3, table, ns)
    return jnp.transpose(out5, (2, 4, 0, 1, 3)).reshape(nb, ns, EMB)
